# baseline (speedup 1.0000x reference)
import jax
import jax.numpy as jnp
from jax import lax
from jax.experimental import pallas as pl
from jax.experimental.pallas import tpu as pltpu
from jax.experimental.pallas import tpu_sc as plsc

NUM_CORES = 2
NUM_SUBCORES = 16
NUM_WORKERS = NUM_CORES * NUM_SUBCORES

BATCH_SIZE = 16384
ROW_DIM = 64
CHUNK = 128
ROWS_PER_WORKER = BATCH_SIZE // NUM_WORKERS
CHUNKS_PER_WORKER = ROWS_PER_WORKER // CHUNK


def _gather_body(table_hbm, idx_hbm, out_hbm, idx_v, rows_v, osem, *gsems):
    wid = lax.axis_index("s") * NUM_CORES + lax.axis_index("c")
    base = wid * ROWS_PER_WORKER
    pltpu.sync_copy(idx_hbm.at[wid], idx_v)
    gathers = [
        pltpu.async_copy(
            table_hbm.at[idx_v.at[j]],
            rows_v.at[pl.ds(j * CHUNK, CHUNK)],
            gsems[j],
        )
        for j in range(CHUNKS_PER_WORKER)
    ]
    writes = []
    for j in range(CHUNKS_PER_WORKER):
        gathers[j].wait()
        writes.append(
            pltpu.async_copy(
                rows_v.at[pl.ds(j * CHUNK, CHUNK)],
                out_hbm.at[pl.ds(base + j * CHUNK, CHUNK)],
                osem,
            )
        )
    for w in writes:
        w.wait()


@jax.jit
def _gather(table, ids):
    mesh = plsc.VectorSubcoreMesh(
        core_axis_name="c", subcore_axis_name="s",
        num_cores=NUM_CORES, num_subcores=NUM_SUBCORES,
    )
    fn = pl.kernel(
        _gather_body,
        out_type=jax.ShapeDtypeStruct((BATCH_SIZE, ROW_DIM), jnp.float32),
        mesh=mesh,
        scratch_types=[
            pltpu.VMEM((CHUNKS_PER_WORKER, CHUNK), jnp.int32),
            pltpu.VMEM((ROWS_PER_WORKER, ROW_DIM), jnp.float32),
            pltpu.SemaphoreType.DMA,
        ] + [pltpu.SemaphoreType.DMA] * CHUNKS_PER_WORKER,
        compiler_params=pltpu.CompilerParams(use_tc_tiling_on_sc=False),
    )
    return fn(table, ids)


def kernel(speaker_ids, embedding_table):
    ids = speaker_ids.astype(jnp.int32).reshape(
        NUM_WORKERS, CHUNKS_PER_WORKER, CHUNK
    )
    return _gather(embedding_table, ids)


# speedup vs baseline: 1.9756x; 1.9756x over previous
import jax
import jax.numpy as jnp
from jax import lax
from jax.experimental import pallas as pl
from jax.experimental.pallas import tpu as pltpu
from jax.experimental.pallas import tpu_sc as plsc

NUM_CORES = 2
NUM_SUBCORES = 16
NUM_WORKERS = NUM_CORES * NUM_SUBCORES

BATCH_SIZE = 16384
ROW_DIM = 64
VOCAB = 100000
DIMS_PER_WORKER = ROW_DIM // NUM_WORKERS
CHUNK = 4096
NUM_CHUNKS = BATCH_SIZE // CHUNK
LANES = 16
VREGS_PER_CHUNK = CHUNK // LANES


def _lookup_body(table_t, ids_hbm, out_t, ids_v, row_v, stage_a, stage_b, sem):
    wid = lax.axis_index("s") * NUM_CORES + lax.axis_index("c")
    pltpu.sync_copy(ids_hbm, ids_v)
    stages = (stage_a, stage_b)
    pending = []
    for r in range(DIMS_PER_WORKER):
        c = wid * DIMS_PER_WORKER + r
        pltpu.sync_copy(table_t.at[c], row_v)
        for k in range(NUM_CHUNKS):
            stage = stages[k % 2]
            if len(pending) >= 2:
                pending.pop(0).wait()

            def gather_chunk(g, _, k=k, stage=stage):
                idx = ids_v[pl.ds(k * CHUNK + g * LANES, LANES)]
                stage[pl.ds(g * LANES, LANES)] = plsc.load_gather(row_v, [idx])
                return 0

            lax.fori_loop(0, VREGS_PER_CHUNK, gather_chunk, 0)
            pending.append(
                pltpu.async_copy(
                    stage, out_t.at[c, pl.ds(k * CHUNK, CHUNK)], sem
                )
            )
    for w in pending:
        w.wait()


@jax.jit
def _lookup(table_t, ids):
    mesh = plsc.VectorSubcoreMesh(
        core_axis_name="c", subcore_axis_name="s",
        num_cores=NUM_CORES, num_subcores=NUM_SUBCORES,
    )
    fn = pl.kernel(
        _lookup_body,
        out_type=jax.ShapeDtypeStruct((ROW_DIM, BATCH_SIZE), jnp.float32),
        mesh=mesh,
        scratch_types=[
            pltpu.VMEM((BATCH_SIZE,), jnp.int32),
            pltpu.VMEM((VOCAB,), jnp.float32),
            pltpu.VMEM((CHUNK,), jnp.float32),
            pltpu.VMEM((CHUNK,), jnp.float32),
            pltpu.SemaphoreType.DMA,
        ],
        compiler_params=pltpu.CompilerParams(needs_layout_passes=False),
    )
    return fn(table_t, ids)


def kernel(speaker_ids, embedding_table):
    ids = speaker_ids.astype(jnp.int32)
    out_t = _lookup(embedding_table.T, ids)
    return out_t.T


# speedup vs baseline: 2.7759x; 1.4051x over previous
import jax
import jax.numpy as jnp
from jax import lax
from jax.experimental import pallas as pl
from jax.experimental.pallas import tpu as pltpu
from jax.experimental.pallas import tpu_sc as plsc

NUM_CORES = 2
NUM_SUBCORES = 16
NUM_WORKERS = NUM_CORES * NUM_SUBCORES

BATCH_SIZE = 16384
ROW_DIM = 64
VOCAB = 100000
DIMS_PER_WORKER = ROW_DIM // NUM_WORKERS
CHUNK = 4096
NUM_CHUNKS = BATCH_SIZE // CHUNK
LANES = 16


def _lookup_body(table_t, ids_hbm, out_t, ids_v, row_v, stage_a, stage_b,
                 isem, rsem, osem):
    wid = lax.axis_index("s") * NUM_CORES + lax.axis_index("c")
    c0 = wid * DIMS_PER_WORKER
    ids_cp = pltpu.async_copy(ids_hbm, ids_v, isem)
    row_cp = pltpu.async_copy(table_t.at[c0], row_v, rsem)
    ids_cp.wait()
    stages = (stage_a, stage_b)
    pending = []
    for r in range(DIMS_PER_WORKER):
        c = c0 + r
        row_cp.wait()
        for k in range(NUM_CHUNKS):
            stage = stages[k % 2]
            if len(pending) >= 2:
                pending.pop(0).wait()

            @plsc.parallel_loop(0, CHUNK, LANES, unroll=8)
            def gather_chunk(g, k=k, stage=stage):
                idx = ids_v[pl.ds(k * CHUNK + g, LANES)]
                stage[pl.ds(g, LANES)] = plsc.load_gather(row_v, [idx])

            pending.append(
                pltpu.async_copy(
                    stage, out_t.at[c, pl.ds(k * CHUNK, CHUNK)], osem
                )
            )
        if r + 1 < DIMS_PER_WORKER:
            row_cp = pltpu.async_copy(table_t.at[c0 + r + 1], row_v, rsem)
    for w in pending:
        w.wait()


@jax.jit
def _lookup(table_t, ids):
    mesh = plsc.VectorSubcoreMesh(
        core_axis_name="c", subcore_axis_name="s",
        num_cores=NUM_CORES, num_subcores=NUM_SUBCORES,
    )
    fn = pl.kernel(
        _lookup_body,
        out_type=jax.ShapeDtypeStruct((ROW_DIM, BATCH_SIZE), jnp.float32),
        mesh=mesh,
        scratch_types=[
            pltpu.VMEM((BATCH_SIZE,), jnp.int32),
            pltpu.VMEM((VOCAB,), jnp.float32),
            pltpu.VMEM((CHUNK,), jnp.float32),
            pltpu.VMEM((CHUNK,), jnp.float32),
            pltpu.SemaphoreType.DMA,
            pltpu.SemaphoreType.DMA,
            pltpu.SemaphoreType.DMA,
        ],
        compiler_params=pltpu.CompilerParams(needs_layout_passes=False),
    )
    return fn(table_t, ids)


def kernel(speaker_ids, embedding_table):
    ids = speaker_ids.astype(jnp.int32)
    out_t = _lookup(embedding_table.T, ids)
    return out_t.T
